# initial kernel scaffold (unmeasured)
import jax
import jax.numpy as jnp
from jax import lax
from jax.experimental import pallas as pl
from jax.experimental.pallas import tpu as pltpu

N_DEV = 4
B, SQ, D_MODEL = 2, 512, 768
HQ_TOTAL, DH = 32, 64
H_LOC = HQ_TOTAL // N_DEV
SKV_LOC = 512
SKV = N_DEV * SKV_LOC
HD_LOC = H_LOC * DH
ROWS = B * SQ
CH = ROWS // N_DEV
BLK = 64

_MESH = pl.DeviceIdType.MESH
_F32 = jnp.float32


def kernel(x, Wq, K_ext, V_ext, Wo):
    def body(x_ref, wq_ref, k_ref, v_ref, wo_ref, out_ref,
             k_full, v_full, q_scr, ctx_scr, p_ref, rs_buf, o4,
             cp_sems, ksend, krecv, vsend, vrecv,
             rs_send, rs_recv, ag_send, ag_recv):
        me = lax.axis_index("i")
        right = (me + 1) % N_DEV

        bsem = pltpu.get_barrier_semaphore()
        for o in range(1, N_DEV):
            pl.semaphore_signal(bsem, inc=1, device_id=((me + o) % N_DEV,),
                                device_id_type=_MESH)
        pl.semaphore_wait(bsem, N_DEV - 1)

        ck = pltpu.make_async_copy(
            k_ref.at[:, :, pl.ds(me * H_LOC, H_LOC), :],
            k_full.at[:, pl.ds(me * SKV_LOC, SKV_LOC), :, :],
            cp_sems.at[0])
        cv = pltpu.make_async_copy(
            v_ref.at[:, :, pl.ds(me * H_LOC, H_LOC), :],
            v_full.at[:, pl.ds(me * SKV_LOC, SKV_LOC), :, :],
            cp_sems.at[1])
        ck.start()
        cv.start()

        kv_rdmas = []
        for o in range(1, N_DEV):
            dst = (me + o) % N_DEV
            rk = pltpu.make_async_remote_copy(
                src_ref=k_ref.at[:, :, pl.ds(dst * H_LOC, H_LOC), :],
                dst_ref=k_full.at[:, pl.ds(me * SKV_LOC, SKV_LOC), :, :],
                send_sem=ksend.at[o], recv_sem=krecv.at[o],
                device_id=(dst,), device_id_type=_MESH)
            rk.start()
            rv = pltpu.make_async_remote_copy(
                src_ref=v_ref.at[:, :, pl.ds(dst * H_LOC, H_LOC), :],
                dst_ref=v_full.at[:, pl.ds(me * SKV_LOC, SKV_LOC), :, :],
                send_sem=vsend.at[o], recv_sem=vrecv.at[o],
                device_id=(dst,), device_id_type=_MESH)
            rv.start()
            kv_rdmas.append((rk, rv))

        for b in range(B):
            q_scr[b] = lax.dot_general(
                x_ref[b], wq_ref[...], (((1,), (0,)), ((), ())),
                preferred_element_type=_F32)

        ck.wait()
        cv.wait()
        for rk, rv in kv_rdmas:
            rk.wait()
            rv.wait()

        qb = lax.broadcasted_iota(jnp.int32, (SQ, SKV), 0) // BLK
        kb = lax.broadcasted_iota(jnp.int32, (SQ, SKV), 1) // BLK
        mask = (qb % 4) == (kb % 4)
        for b in range(B):
            for h in range(H_LOC):
                q = q_scr[b, :, h * DH:(h + 1) * DH]
                k = k_full[b, :, h, :]
                v = v_full[b, :, h, :]
                s = lax.dot_general(q, k, (((1,), (1,)), ((), ())),
                                    preferred_element_type=_F32) * 0.125
                s = jnp.where(mask, s, -1e9)
                m = jnp.max(s, axis=1, keepdims=True)
                e = jnp.exp(s - m)
                w = e / jnp.sum(e, axis=1, keepdims=True)
                ctx_scr[b, :, h * DH:(h + 1) * DH] = lax.dot_general(
                    w, v, (((1,), (0,)), ((), ())),
                    preferred_element_type=_F32)

        for b in range(B):
            pb = lax.dot_general(ctx_scr[b], wo_ref[...],
                                 (((1,), (0,)), ((), ())),
                                 preferred_element_type=_F32)
            p_ref[2 * b] = pb[:CH]
            p_ref[2 * b + 1] = pb[CH:]

        for t in range(N_DEV - 1):
            c_send = (me - t) % N_DEV
            rdma = pltpu.make_async_remote_copy(
                src_ref=p_ref.at[c_send], dst_ref=rs_buf.at[t],
                send_sem=rs_send.at[t], recv_sem=rs_recv.at[t],
                device_id=(right,), device_id_type=_MESH)
            rdma.start()
            rdma.wait()
            c_acc = (me - t - 1) % N_DEV
            p_ref[c_acc] = p_ref[c_acc] + rs_buf[t]

        c_own = (me + 1) % N_DEV
        o4[c_own] = p_ref[c_own]
        for t in range(N_DEV - 1):
            c_send = (me + 1 - t) % N_DEV
            rdma = pltpu.make_async_remote_copy(
                src_ref=o4.at[c_send], dst_ref=o4.at[c_send],
                send_sem=ag_send.at[t], recv_sem=ag_recv.at[t],
                device_id=(right,), device_id_type=_MESH)
            rdma.start()
            rdma.wait()

        for c in range(N_DEV):
            out_ref[c // 2, pl.ds((c % 2) * CH, CH), :] = o4[c]

    vmem = pl.BlockSpec(memory_space=pltpu.MemorySpace.VMEM)
    hbm = pl.BlockSpec(memory_space=pltpu.MemorySpace.ANY)
    return pl.pallas_call(
        body,
        out_shape=jax.ShapeDtypeStruct((B, SQ, D_MODEL), _F32),
        in_specs=[vmem, vmem, hbm, hbm, vmem],
        out_specs=vmem,
        scratch_shapes=[
            pltpu.VMEM((B, SKV, H_LOC, DH), _F32),
            pltpu.VMEM((B, SKV, H_LOC, DH), _F32),
            pltpu.VMEM((B, SQ, HD_LOC), _F32),
            pltpu.VMEM((B, SQ, HD_LOC), _F32),
            pltpu.VMEM((N_DEV, CH, D_MODEL), _F32),
            pltpu.VMEM((N_DEV - 1, CH, D_MODEL), _F32),
            pltpu.VMEM((N_DEV, CH, D_MODEL), _F32),
            pltpu.SemaphoreType.DMA((2,)),
            pltpu.SemaphoreType.DMA((N_DEV,)),
            pltpu.SemaphoreType.DMA((N_DEV,)),
            pltpu.SemaphoreType.DMA((N_DEV,)),
            pltpu.SemaphoreType.DMA((N_DEV,)),
            pltpu.SemaphoreType.DMA((N_DEV - 1,)),
            pltpu.SemaphoreType.DMA((N_DEV - 1,)),
            pltpu.SemaphoreType.DMA((N_DEV - 1,)),
            pltpu.SemaphoreType.DMA((N_DEV - 1,)),
        ],
        compiler_params=pltpu.CompilerParams(collective_id=0),
    )(x, Wq, K_ext, V_ext, Wo)


# baseline (device time: 328229 ns/iter reference)
import jax
import jax.numpy as jnp
from jax import lax
from jax.experimental import pallas as pl
from jax.experimental.pallas import tpu as pltpu

N_DEV = 4
B, SQ, D_MODEL = 2, 512, 768
HQ_TOTAL, DH = 32, 64
H_LOC = HQ_TOTAL // N_DEV
SKV_LOC = 512
SKV = N_DEV * SKV_LOC
HD_LOC = H_LOC * DH
ROWS = B * SQ
CH = ROWS // N_DEV
BLK = 64

_MESH = pl.DeviceIdType.MESH
_F32 = jnp.float32

PHASES = ("barrier", "a2a", "attn", "full")


def _build(phase="full", collective_id=0):
    assert phase in PHASES

    def body(x_ref, wq_ref, k_ref, v_ref, wo_ref, out_ref,
             k_full, v_full, q_scr, ctx_scr, p_ref, rs_buf, o4,
             cp_sems, ksend, krecv, vsend, vrecv,
             rs_send, rs_recv, ag_send, ag_recv):
        me = lax.axis_index("i")
        right = (me + 1) % N_DEV

        bsem = pltpu.get_barrier_semaphore()
        for o in range(1, N_DEV):
            pl.semaphore_signal(bsem, inc=1, device_id=((me + o) % N_DEV,),
                                device_id_type=_MESH)
        pl.semaphore_wait(bsem, N_DEV - 1)

        if phase == "barrier":
            out_ref[...] = jnp.zeros_like(out_ref)
            return

        ck = pltpu.make_async_copy(
            k_ref.at[:, :, pl.ds(me * H_LOC, H_LOC), :],
            k_full.at[:, pl.ds(me * SKV_LOC, SKV_LOC), :, :],
            cp_sems.at[0])
        cv = pltpu.make_async_copy(
            v_ref.at[:, :, pl.ds(me * H_LOC, H_LOC), :],
            v_full.at[:, pl.ds(me * SKV_LOC, SKV_LOC), :, :],
            cp_sems.at[1])
        ck.start()
        cv.start()

        kv_rdmas = []
        for o in range(1, N_DEV):
            dst = (me + o) % N_DEV
            rk = pltpu.make_async_remote_copy(
                src_ref=k_ref.at[:, :, pl.ds(dst * H_LOC, H_LOC), :],
                dst_ref=k_full.at[:, pl.ds(me * SKV_LOC, SKV_LOC), :, :],
                send_sem=ksend.at[o], recv_sem=krecv.at[o],
                device_id=(dst,), device_id_type=_MESH)
            rk.start()
            rv = pltpu.make_async_remote_copy(
                src_ref=v_ref.at[:, :, pl.ds(dst * H_LOC, H_LOC), :],
                dst_ref=v_full.at[:, pl.ds(me * SKV_LOC, SKV_LOC), :, :],
                send_sem=vsend.at[o], recv_sem=vrecv.at[o],
                device_id=(dst,), device_id_type=_MESH)
            rv.start()
            kv_rdmas.append((rk, rv))

        for b in range(B):
            q_scr[b] = lax.dot_general(
                x_ref[b], wq_ref[...], (((1,), (0,)), ((), ())),
                preferred_element_type=_F32)

        ck.wait()
        cv.wait()
        for rk, rv in kv_rdmas:
            rk.wait()
            rv.wait()

        if phase == "a2a":
            out_ref[...] = jnp.zeros_like(out_ref)
            out_ref[0, :, :DH] = k_full[0, :SQ, 0, :]
            return

        qb = lax.broadcasted_iota(jnp.int32, (SQ, SKV), 0) // BLK
        kb = lax.broadcasted_iota(jnp.int32, (SQ, SKV), 1) // BLK
        mask = (qb % 4) == (kb % 4)
        for b in range(B):
            for h in range(H_LOC):
                q = q_scr[b, :, h * DH:(h + 1) * DH]
                k = k_full[b, :, h, :]
                v = v_full[b, :, h, :]
                s = lax.dot_general(q, k, (((1,), (1,)), ((), ())),
                                    preferred_element_type=_F32) * 0.125
                s = jnp.where(mask, s, -1e9)
                m = jnp.max(s, axis=1, keepdims=True)
                e = jnp.exp(s - m)
                w = e / jnp.sum(e, axis=1, keepdims=True)
                ctx_scr[b, :, h * DH:(h + 1) * DH] = lax.dot_general(
                    w, v, (((1,), (0,)), ((), ())),
                    preferred_element_type=_F32)

        for b in range(B):
            pb = lax.dot_general(ctx_scr[b], wo_ref[...],
                                 (((1,), (0,)), ((), ())),
                                 preferred_element_type=_F32)
            p_ref[2 * b] = pb[:CH]
            p_ref[2 * b + 1] = pb[CH:]

        if phase == "attn":
            for c in range(N_DEV):
                out_ref[c // 2, pl.ds((c % 2) * CH, CH), :] = p_ref[c]
            return

        for t in range(N_DEV - 1):
            c_send = (me - t) % N_DEV
            rdma = pltpu.make_async_remote_copy(
                src_ref=p_ref.at[c_send], dst_ref=rs_buf.at[t],
                send_sem=rs_send.at[t], recv_sem=rs_recv.at[t],
                device_id=(right,), device_id_type=_MESH)
            rdma.start()
            rdma.wait()
            c_acc = (me - t - 1) % N_DEV
            p_ref[c_acc] = p_ref[c_acc] + rs_buf[t]

        c_own = (me + 1) % N_DEV
        o4[c_own] = p_ref[c_own]
        for t in range(N_DEV - 1):
            c_send = (me + 1 - t) % N_DEV
            rdma = pltpu.make_async_remote_copy(
                src_ref=o4.at[c_send], dst_ref=o4.at[c_send],
                send_sem=ag_send.at[t], recv_sem=ag_recv.at[t],
                device_id=(right,), device_id_type=_MESH)
            rdma.start()
            rdma.wait()

        for c in range(N_DEV):
            out_ref[c // 2, pl.ds((c % 2) * CH, CH), :] = o4[c]

    vmem = pl.BlockSpec(memory_space=pltpu.MemorySpace.VMEM)
    hbm = pl.BlockSpec(memory_space=pltpu.MemorySpace.HBM)

    def kernel(x, Wq, K_ext, V_ext, Wo):
        return pl.pallas_call(
            body,
            out_shape=jax.ShapeDtypeStruct((B, SQ, D_MODEL), _F32),
            in_specs=[vmem, vmem, hbm, hbm, vmem],
            out_specs=vmem,
            scratch_shapes=[
                pltpu.VMEM((B, SKV, H_LOC, DH), _F32),
                pltpu.VMEM((B, SKV, H_LOC, DH), _F32),
                pltpu.VMEM((B, SQ, HD_LOC), _F32),
                pltpu.VMEM((B, SQ, HD_LOC), _F32),
                pltpu.VMEM((N_DEV, CH, D_MODEL), _F32),
                pltpu.VMEM((N_DEV - 1, CH, D_MODEL), _F32),
                pltpu.VMEM((N_DEV, CH, D_MODEL), _F32),
                pltpu.SemaphoreType.DMA((2,)),
                pltpu.SemaphoreType.DMA((N_DEV,)),
                pltpu.SemaphoreType.DMA((N_DEV,)),
                pltpu.SemaphoreType.DMA((N_DEV,)),
                pltpu.SemaphoreType.DMA((N_DEV,)),
                pltpu.SemaphoreType.DMA((N_DEV - 1,)),
                pltpu.SemaphoreType.DMA((N_DEV - 1,)),
                pltpu.SemaphoreType.DMA((N_DEV - 1,)),
                pltpu.SemaphoreType.DMA((N_DEV - 1,)),
            ],
            compiler_params=pltpu.CompilerParams(
                collective_id=collective_id,
                vmem_limit_bytes=60 * 1024 * 1024),
        )(x, Wq, K_ext, V_ext, Wo)

    return kernel


kernel = _build("full")


# device time: 318870 ns/iter; 1.0294x vs baseline; 1.0294x over previous
import jax
import jax.numpy as jnp
from jax import lax
from jax.experimental import pallas as pl
from jax.experimental.pallas import tpu as pltpu

N_DEV = 4
B, SQ, D_MODEL = 2, 512, 768
HQ_TOTAL, DH = 32, 64
H_LOC = HQ_TOTAL // N_DEV
SKV_LOC = 512
SKV = N_DEV * SKV_LOC
HD_LOC = H_LOC * DH
ROWS = B * SQ
CH = ROWS // N_DEV
BLK = 64

_MESH = pl.DeviceIdType.MESH
_F32 = jnp.float32

PHASES = ("barrier", "a2a", "attn", "full")


def _build(phase="full", collective_id=0):
    assert phase in PHASES

    def body(x_ref, wq_ref, kv_ref, wo_ref, out_ref,
             kv_recv, q_scr, ctx_scr, p_ref, rs_buf, o4,
             cp_sem, kv_send, kv_rsem,
             rs_send, rs_recv, ag_send, ag_recv):
        me = lax.axis_index("i")
        right = (me + 1) % N_DEV

        bsem = pltpu.get_barrier_semaphore()
        for o in range(1, N_DEV):
            pl.semaphore_signal(bsem, inc=1, device_id=((me + o) % N_DEV,),
                                device_id_type=_MESH)
        pl.semaphore_wait(bsem, N_DEV - 1)

        if phase == "barrier":
            out_ref[...] = jnp.zeros_like(out_ref)
            return

        cp = pltpu.make_async_copy(
            kv_ref.at[pl.ds(me * H_LOC, H_LOC)], kv_recv.at[me], cp_sem)
        cp.start()

        kv_rdmas = []
        for o in range(1, N_DEV):
            dst = (me + o) % N_DEV
            r = pltpu.make_async_remote_copy(
                src_ref=kv_ref.at[pl.ds(dst * H_LOC, H_LOC)],
                dst_ref=kv_recv.at[me],
                send_sem=kv_send.at[o], recv_sem=kv_rsem.at[o],
                device_id=(dst,), device_id_type=_MESH)
            r.start()
            kv_rdmas.append(r)

        for b in range(B):
            q_scr[b] = lax.dot_general(
                x_ref[b], wq_ref[...], (((1,), (0,)), ((), ())),
                preferred_element_type=_F32)

        cp.wait()
        for r in kv_rdmas:
            r.wait()

        if phase == "a2a":
            out_ref[...] = jnp.zeros_like(out_ref)
            out_ref[0, :, :DH] = kv_recv[0, 0, 0, 0, :, :]
            return

        qb = lax.broadcasted_iota(jnp.int32, (SQ, SKV), 0) // BLK
        kb = lax.broadcasted_iota(jnp.int32, (SQ, SKV), 1) // BLK
        mask = (qb % 4) == (kb % 4)
        for b in range(B):
            for h in range(H_LOC):
                q = q_scr[b, :, h * DH:(h + 1) * DH]
                k = jnp.concatenate(
                    [kv_recv[s, h, 0, b] for s in range(N_DEV)], axis=0)
                v = jnp.concatenate(
                    [kv_recv[s, h, 1, b] for s in range(N_DEV)], axis=0)
                s = lax.dot_general(q, k, (((1,), (1,)), ((), ())),
                                    preferred_element_type=_F32) * 0.125
                s = jnp.where(mask, s, -1e9)
                m = jnp.max(s, axis=1, keepdims=True)
                e = jnp.exp(s - m)
                w = e / jnp.sum(e, axis=1, keepdims=True)
                ctx_scr[b, :, h * DH:(h + 1) * DH] = lax.dot_general(
                    w, v, (((1,), (0,)), ((), ())),
                    preferred_element_type=_F32)

        for b in range(B):
            pb = lax.dot_general(ctx_scr[b], wo_ref[...],
                                 (((1,), (0,)), ((), ())),
                                 preferred_element_type=_F32)
            p_ref[2 * b] = pb[:CH]
            p_ref[2 * b + 1] = pb[CH:]

        if phase == "attn":
            for c in range(N_DEV):
                out_ref[c // 2, pl.ds((c % 2) * CH, CH), :] = p_ref[c]
            return

        for t in range(N_DEV - 1):
            c_send = (me - t) % N_DEV
            rdma = pltpu.make_async_remote_copy(
                src_ref=p_ref.at[c_send], dst_ref=rs_buf.at[t],
                send_sem=rs_send.at[t], recv_sem=rs_recv.at[t],
                device_id=(right,), device_id_type=_MESH)
            rdma.start()
            rdma.wait()
            c_acc = (me - t - 1) % N_DEV
            p_ref[c_acc] = p_ref[c_acc] + rs_buf[t]

        c_own = (me + 1) % N_DEV
        o4[c_own] = p_ref[c_own]
        for t in range(N_DEV - 1):
            c_send = (me + 1 - t) % N_DEV
            rdma = pltpu.make_async_remote_copy(
                src_ref=o4.at[c_send], dst_ref=o4.at[c_send],
                send_sem=ag_send.at[t], recv_sem=ag_recv.at[t],
                device_id=(right,), device_id_type=_MESH)
            rdma.start()
            rdma.wait()

        for c in range(N_DEV):
            out_ref[c // 2, pl.ds((c % 2) * CH, CH), :] = o4[c]

    vmem = pl.BlockSpec(memory_space=pltpu.MemorySpace.VMEM)
    hbm = pl.BlockSpec(memory_space=pltpu.MemorySpace.HBM)

    grid_kernel = pl.pallas_call(
        body,
        out_shape=jax.ShapeDtypeStruct((B, SQ, D_MODEL), _F32),
        in_specs=[vmem, vmem, hbm, vmem],
        out_specs=vmem,
        scratch_shapes=[
            pltpu.VMEM((N_DEV, H_LOC, 2, B, SKV_LOC, DH), _F32),
            pltpu.VMEM((B, SQ, HD_LOC), _F32),
            pltpu.VMEM((B, SQ, HD_LOC), _F32),
            pltpu.VMEM((N_DEV, CH, D_MODEL), _F32),
            pltpu.VMEM((N_DEV - 1, CH, D_MODEL), _F32),
            pltpu.VMEM((N_DEV, CH, D_MODEL), _F32),
            pltpu.SemaphoreType.DMA,
            pltpu.SemaphoreType.DMA((N_DEV,)),
            pltpu.SemaphoreType.DMA((N_DEV,)),
            pltpu.SemaphoreType.DMA((N_DEV - 1,)),
            pltpu.SemaphoreType.DMA((N_DEV - 1,)),
            pltpu.SemaphoreType.DMA((N_DEV - 1,)),
            pltpu.SemaphoreType.DMA((N_DEV - 1,)),
        ],
        compiler_params=pltpu.CompilerParams(
            collective_id=collective_id,
            vmem_limit_bytes=60 * 1024 * 1024),
    )

    def kernel(x, Wq, K_ext, V_ext, Wo):
        k_t = jnp.transpose(K_ext, (2, 0, 1, 3))
        v_t = jnp.transpose(V_ext, (2, 0, 1, 3))
        kv = jnp.stack([k_t, v_t], axis=1)
        return grid_kernel(x, Wq, kv, Wo)

    return kernel


kernel = _build("full")


# device time: 192398 ns/iter; 1.7060x vs baseline; 1.6573x over previous
import jax
import jax.numpy as jnp
from jax import lax
from jax.experimental import pallas as pl
from jax.experimental.pallas import tpu as pltpu

N_DEV = 4
B, SQ, D_MODEL = 2, 512, 768
HQ_TOTAL, DH = 32, 64
H_LOC = HQ_TOTAL // N_DEV
SKV_LOC = 512
SKV = N_DEV * SKV_LOC
HD_LOC = H_LOC * DH
ROWS = B * SQ
CH = ROWS // N_DEV
BLK = 64

_MESH = pl.DeviceIdType.MESH
_F32 = jnp.float32
_BF16 = jnp.bfloat16

PHASES = ("barrier", "a2a", "attn", "full")


def _build(phase="full", collective_id=0):
    assert phase in PHASES

    def body(x_ref, wq_ref, kv_ref, wo_ref, out_ref,
             kv_recv, q_scr, ctx_scr, p_ref, rs_buf, o4,
             cp_sem, kv_send, kv_rsem,
             rs_send, rs_recv, ag_send, ag_recv):
        me = lax.axis_index("i")
        right = (me + 1) % N_DEV

        bsem = pltpu.get_barrier_semaphore()
        for o in range(1, N_DEV):
            pl.semaphore_signal(bsem, inc=1, device_id=((me + o) % N_DEV,),
                                device_id_type=_MESH)
        pl.semaphore_wait(bsem, N_DEV - 1)

        if phase == "barrier":
            out_ref[...] = jnp.zeros_like(out_ref)
            return

        cp = pltpu.make_async_copy(
            kv_ref.at[pl.ds(me * H_LOC, H_LOC)], kv_recv.at[me], cp_sem)
        cp.start()

        kv_rdmas = []
        for o in range(1, N_DEV):
            dst = (me + o) % N_DEV
            r = pltpu.make_async_remote_copy(
                src_ref=kv_ref.at[pl.ds(dst * H_LOC, H_LOC)],
                dst_ref=kv_recv.at[me],
                send_sem=kv_send.at[o], recv_sem=kv_rsem.at[o],
                device_id=(dst,), device_id_type=_MESH)
            r.start()
            kv_rdmas.append(r)

        for b in range(B):
            q_scr[b] = lax.dot_general(
                x_ref[b], wq_ref[...], (((1,), (0,)), ((), ())),
                preferred_element_type=_F32)

        cp.wait()
        for r in kv_rdmas:
            r.wait()

        if phase == "a2a":
            out_ref[...] = jnp.zeros_like(out_ref)
            out_ref[0, :, :DH] = kv_recv[0, 0, 0, 0, :, :]
            return

        qb = lax.broadcasted_iota(jnp.int32, (SQ, SKV), 0) // BLK
        kb = lax.broadcasted_iota(jnp.int32, (SQ, SKV), 1) // BLK
        mask = (qb % 4) == (kb % 4)
        for b in range(B):
            for h in range(H_LOC):
                q = q_scr[b, :, h * DH:(h + 1) * DH].astype(_BF16)
                k = jnp.concatenate(
                    [kv_recv[s, h, 0, b] for s in range(N_DEV)], axis=0)
                v = jnp.concatenate(
                    [kv_recv[s, h, 1, b] for s in range(N_DEV)], axis=0)
                s = lax.dot_general(q, k, (((1,), (1,)), ((), ())),
                                    preferred_element_type=_F32) * 0.125
                s = jnp.where(mask, s, -1e9)
                m = jnp.max(s, axis=1, keepdims=True)
                e = jnp.exp(s - m)
                w = (e / jnp.sum(e, axis=1, keepdims=True)).astype(_BF16)
                ctx_scr[b, :, h * DH:(h + 1) * DH] = lax.dot_general(
                    w, v, (((1,), (0,)), ((), ())),
                    preferred_element_type=_F32)

        for b in range(B):
            pb = lax.dot_general(ctx_scr[b], wo_ref[...],
                                 (((1,), (0,)), ((), ())),
                                 preferred_element_type=_F32)
            pb = pb.astype(_BF16)
            p_ref[2 * b] = pb[:CH]
            p_ref[2 * b + 1] = pb[CH:]

        if phase == "attn":
            for c in range(N_DEV):
                out_ref[c // 2, pl.ds((c % 2) * CH, CH), :] = p_ref[c].astype(_F32)
            return

        for t in range(N_DEV - 1):
            c_send = (me - t) % N_DEV
            rdma = pltpu.make_async_remote_copy(
                src_ref=p_ref.at[c_send], dst_ref=rs_buf.at[t],
                send_sem=rs_send.at[t], recv_sem=rs_recv.at[t],
                device_id=(right,), device_id_type=_MESH)
            rdma.start()
            rdma.wait()
            c_acc = (me - t - 1) % N_DEV
            p_ref[c_acc] = p_ref[c_acc] + rs_buf[t]

        c_own = (me + 1) % N_DEV
        o4[c_own] = p_ref[c_own]
        for t in range(N_DEV - 1):
            c_send = (me + 1 - t) % N_DEV
            rdma = pltpu.make_async_remote_copy(
                src_ref=o4.at[c_send], dst_ref=o4.at[c_send],
                send_sem=ag_send.at[t], recv_sem=ag_recv.at[t],
                device_id=(right,), device_id_type=_MESH)
            rdma.start()
            rdma.wait()

        for c in range(N_DEV):
            out_ref[c // 2, pl.ds((c % 2) * CH, CH), :] = o4[c].astype(_F32)

    vmem = pl.BlockSpec(memory_space=pltpu.MemorySpace.VMEM)
    hbm = pl.BlockSpec(memory_space=pltpu.MemorySpace.HBM)

    grid_kernel = pl.pallas_call(
        body,
        out_shape=jax.ShapeDtypeStruct((B, SQ, D_MODEL), _F32),
        in_specs=[vmem, vmem, vmem, vmem],
        out_specs=vmem,
        scratch_shapes=[
            pltpu.VMEM((N_DEV, H_LOC, 2, B, SKV_LOC, DH), _BF16),
            pltpu.VMEM((B, SQ, HD_LOC), _F32),
            pltpu.VMEM((B, SQ, HD_LOC), _F32),
            pltpu.VMEM((N_DEV, CH, D_MODEL), _BF16),
            pltpu.VMEM((N_DEV - 1, CH, D_MODEL), _BF16),
            pltpu.VMEM((N_DEV, CH, D_MODEL), _BF16),
            pltpu.SemaphoreType.DMA,
            pltpu.SemaphoreType.DMA((N_DEV,)),
            pltpu.SemaphoreType.DMA((N_DEV,)),
            pltpu.SemaphoreType.DMA((N_DEV - 1,)),
            pltpu.SemaphoreType.DMA((N_DEV - 1,)),
            pltpu.SemaphoreType.DMA((N_DEV - 1,)),
            pltpu.SemaphoreType.DMA((N_DEV - 1,)),
        ],
        compiler_params=pltpu.CompilerParams(
            collective_id=collective_id,
            vmem_limit_bytes=60 * 1024 * 1024),
    )

    def kernel(x, Wq, K_ext, V_ext, Wo):
        k_t = jnp.transpose(K_ext, (2, 0, 1, 3))
        v_t = jnp.transpose(V_ext, (2, 0, 1, 3))
        kv = jnp.stack([k_t, v_t], axis=1).astype(_BF16)
        return grid_kernel(x, Wq, kv, Wo)

    return kernel


kernel = _build("full")


# device time: 128726 ns/iter; 2.5498x vs baseline; 1.4946x over previous
import jax
import jax.numpy as jnp
from jax import lax
from jax.experimental import pallas as pl
from jax.experimental.pallas import tpu as pltpu

N_DEV = 4
B, SQ, D_MODEL = 2, 512, 768
HQ_TOTAL, DH = 32, 64
H_LOC = HQ_TOTAL // N_DEV
SKV_LOC = 512
SKV = N_DEV * SKV_LOC
HD_LOC = H_LOC * DH
ROWS = B * SQ
CH = ROWS // N_DEV
BLK = 64

_MESH = pl.DeviceIdType.MESH
_F32 = jnp.float32
_BF16 = jnp.bfloat16

PHASES = ("barrier", "a2a", "attn", "full")


def _build(phase="full", collective_id=0):
    assert phase in PHASES

    def body(x_ref, wq_ref, kv_ref, wo_ref, out_ref,
             kv_recv, q_scr, ctx_scr, p_ref, rs_buf, o4,
             cp_sem, kv_send, kv_rsem,
             rs_send, rs_recv, ag_send, ag_recv):
        me = lax.axis_index("i")
        right = (me + 1) % N_DEV

        bsem = pltpu.get_barrier_semaphore()
        for o in range(1, N_DEV):
            pl.semaphore_signal(bsem, inc=1, device_id=((me + o) % N_DEV,),
                                device_id_type=_MESH)
        pl.semaphore_wait(bsem, N_DEV - 1)

        if phase == "barrier":
            out_ref[...] = jnp.zeros_like(out_ref)
            return

        cp = pltpu.make_async_copy(
            kv_ref.at[pl.ds(me * H_LOC, H_LOC)], kv_recv.at[me], cp_sem)
        cp.start()

        kv_rdmas = []
        for o in range(1, N_DEV):
            dst = (me + o) % N_DEV
            r = pltpu.make_async_remote_copy(
                src_ref=kv_ref.at[pl.ds(dst * H_LOC, H_LOC)],
                dst_ref=kv_recv.at[me],
                send_sem=kv_send.at[o], recv_sem=kv_rsem.at[o],
                device_id=(dst,), device_id_type=_MESH)
            r.start()
            kv_rdmas.append(r)

        for b in range(B):
            q_scr[b] = lax.dot_general(
                x_ref[b], wq_ref[...], (((1,), (0,)), ((), ())),
                preferred_element_type=_F32)

        if phase == "a2a":
            cp.wait()
            for r in kv_rdmas:
                r.wait()
            out_ref[...] = jnp.zeros_like(out_ref)
            out_ref[0, :, :DH] = kv_recv[0, 0, 0, :, :DH].astype(_F32)
            return

        qb0 = lax.broadcasted_iota(jnp.int32, (SQ, SKV_LOC), 0) // BLK
        kb0 = lax.broadcasted_iota(jnp.int32, (SQ, SKV_LOC), 1) // BLK
        mask_s = (qb0 % 4) == (kb0 % 4)

        l_run = [[None] * H_LOC for _ in range(B)]
        acc = [[None] * H_LOC for _ in range(B)]

        def process_slot(slot):
            for b in range(B):
                for h in range(H_LOC):
                    q = q_scr[b, :, h * DH:(h + 1) * DH].astype(_BF16)
                    k = kv_recv[slot, h, b, :, :DH]
                    v = kv_recv[slot, h, b, :, DH:]
                    sb = lax.dot_general(q, k, (((1,), (1,)), ((), ())),
                                         preferred_element_type=_F32) * 0.125
                    e = jnp.where(mask_s, jnp.exp(sb), 0.0)
                    pv = lax.dot_general(
                        e.astype(_BF16), v, (((1,), (0,)), ((), ())),
                        preferred_element_type=_F32)
                    ps = jnp.sum(e, axis=1)
                    if l_run[b][h] is None:
                        l_run[b][h], acc[b][h] = ps, pv
                    else:
                        l_run[b][h] = l_run[b][h] + ps
                        acc[b][h] = acc[b][h] + pv

        cp.wait()
        process_slot(me)
        for o in (1, 3, 2):
            kv_rdmas[o - 1].wait()
            process_slot((me - o) % N_DEV)

        for b in range(B):
            for h in range(H_LOC):
                ctx_scr[b, :, h * DH:(h + 1) * DH] = (
                    acc[b][h] / l_run[b][h][:, None])

        for b in range(B):
            pb = lax.dot_general(ctx_scr[b], wo_ref[...],
                                 (((1,), (0,)), ((), ())),
                                 preferred_element_type=_F32)
            pb = pb.astype(_BF16)
            p_ref[2 * b] = pb[:CH]
            p_ref[2 * b + 1] = pb[CH:]

        if phase == "attn":
            for c in range(N_DEV):
                out_ref[c // 2, pl.ds((c % 2) * CH, CH), :] = p_ref[c].astype(_F32)
            return

        for t in range(N_DEV - 1):
            c_send = (me - t) % N_DEV
            rdma = pltpu.make_async_remote_copy(
                src_ref=p_ref.at[c_send], dst_ref=rs_buf.at[t],
                send_sem=rs_send.at[t], recv_sem=rs_recv.at[t],
                device_id=(right,), device_id_type=_MESH)
            rdma.start()
            rdma.wait()
            c_acc = (me - t - 1) % N_DEV
            p_ref[c_acc] = p_ref[c_acc] + rs_buf[t]

        c_own = (me + 1) % N_DEV
        o4[c_own] = p_ref[c_own]
        for t in range(N_DEV - 1):
            c_send = (me + 1 - t) % N_DEV
            rdma = pltpu.make_async_remote_copy(
                src_ref=o4.at[c_send], dst_ref=o4.at[c_send],
                send_sem=ag_send.at[t], recv_sem=ag_recv.at[t],
                device_id=(right,), device_id_type=_MESH)
            rdma.start()
            rdma.wait()

        for c in range(N_DEV):
            out_ref[c // 2, pl.ds((c % 2) * CH, CH), :] = o4[c].astype(_F32)

    vmem = pl.BlockSpec(memory_space=pltpu.MemorySpace.VMEM)
    hbm = pl.BlockSpec(memory_space=pltpu.MemorySpace.HBM)

    grid_kernel = pl.pallas_call(
        body,
        out_shape=jax.ShapeDtypeStruct((B, SQ, D_MODEL), _F32),
        in_specs=[vmem, vmem, vmem, vmem],
        out_specs=vmem,
        scratch_shapes=[
            pltpu.VMEM((N_DEV, H_LOC, B, SKV_LOC, 2 * DH), _BF16),
            pltpu.VMEM((B, SQ, HD_LOC), _F32),
            pltpu.VMEM((B, SQ, HD_LOC), _F32),
            pltpu.VMEM((N_DEV, CH, D_MODEL), _BF16),
            pltpu.VMEM((N_DEV - 1, CH, D_MODEL), _BF16),
            pltpu.VMEM((N_DEV, CH, D_MODEL), _BF16),
            pltpu.SemaphoreType.DMA,
            pltpu.SemaphoreType.DMA((N_DEV,)),
            pltpu.SemaphoreType.DMA((N_DEV,)),
            pltpu.SemaphoreType.DMA((N_DEV - 1,)),
            pltpu.SemaphoreType.DMA((N_DEV - 1,)),
            pltpu.SemaphoreType.DMA((N_DEV - 1,)),
            pltpu.SemaphoreType.DMA((N_DEV - 1,)),
        ],
        compiler_params=pltpu.CompilerParams(
            collective_id=collective_id,
            vmem_limit_bytes=60 * 1024 * 1024),
    )

    def kernel(x, Wq, K_ext, V_ext, Wo):
        k_t = jnp.transpose(K_ext, (2, 0, 1, 3))
        v_t = jnp.transpose(V_ext, (2, 0, 1, 3))
        kv = jnp.concatenate([k_t, v_t], axis=-1).astype(_BF16)
        return grid_kernel(x, Wq, kv, Wo)

    return kernel


kernel = _build("full")


# device time: 114099 ns/iter; 2.8767x vs baseline; 1.1282x over previous
import jax
import jax.numpy as jnp
from jax import lax
from jax.experimental import pallas as pl
from jax.experimental.pallas import tpu as pltpu

N_DEV = 4
B, SQ, D_MODEL = 2, 512, 768
HQ_TOTAL, DH = 32, 64
H_LOC = HQ_TOTAL // N_DEV
SKV_LOC = 512
SKV = N_DEV * SKV_LOC
HD_LOC = H_LOC * DH
ROWS = B * SQ
CH = ROWS // N_DEV
BLK = 64

_MESH = pl.DeviceIdType.MESH
_F32 = jnp.float32
_BF16 = jnp.bfloat16

PHASES = ("barrier", "a2a", "attn", "full")


def _build(phase="full", collective_id=0):
    assert phase in PHASES

    def body(x_ref, wq_ref, kv_ref, wo_ref, out_ref,
             kv_recv, q_scr, ctx_scr, p_ref, rs_buf, o4,
             cp_sem, kv_send, kv_rsem,
             rs_send, rs_recv, ag_send, ag_rsem):
        me = lax.axis_index("i")
        right = (me + 1) % N_DEV

        bsem = pltpu.get_barrier_semaphore()
        for o in range(1, N_DEV):
            pl.semaphore_signal(bsem, inc=1, device_id=((me + o) % N_DEV,),
                                device_id_type=_MESH)
        pl.semaphore_wait(bsem, N_DEV - 1)

        if phase == "barrier":
            out_ref[...] = jnp.zeros_like(out_ref)
            return

        cp = pltpu.make_async_copy(
            kv_ref.at[pl.ds(me * H_LOC, H_LOC)], kv_recv.at[me], cp_sem)
        cp.start()

        kv_rdmas = []
        for o in range(1, N_DEV):
            dst = (me + o) % N_DEV
            r = pltpu.make_async_remote_copy(
                src_ref=kv_ref.at[pl.ds(dst * H_LOC, H_LOC)],
                dst_ref=kv_recv.at[me],
                send_sem=kv_send.at[o], recv_sem=kv_rsem.at[o],
                device_id=(dst,), device_id_type=_MESH)
            r.start()
            kv_rdmas.append(r)

        for b in range(B):
            q_scr[b] = lax.dot_general(
                x_ref[b], wq_ref[...], (((1,), (0,)), ((), ())),
                preferred_element_type=_F32)

        if phase == "a2a":
            cp.wait()
            for r in kv_rdmas:
                r.wait()
            out_ref[...] = jnp.zeros_like(out_ref)
            out_ref[0, :, :DH] = kv_recv[0, 0, 0, :, :DH].astype(_F32)
            return

        qb0 = lax.broadcasted_iota(jnp.int32, (SQ, SKV_LOC), 0) // BLK
        kb0 = lax.broadcasted_iota(jnp.int32, (SQ, SKV_LOC), 1) // BLK
        mask_s = (qb0 % 4) == (kb0 % 4)

        l_run = [[None] * H_LOC for _ in range(B)]
        acc = [[None] * H_LOC for _ in range(B)]

        def process_slot(slot):
            for b in range(B):
                for h in range(H_LOC):
                    q = q_scr[b, :, h * DH:(h + 1) * DH].astype(_BF16)
                    k = kv_recv[slot, h, b, :, :DH]
                    v = kv_recv[slot, h, b, :, DH:]
                    sb = lax.dot_general(q, k, (((1,), (1,)), ((), ())),
                                         preferred_element_type=_F32) * 0.125
                    e = jnp.where(mask_s, jnp.exp(sb), 0.0)
                    pv = lax.dot_general(
                        e.astype(_BF16), v, (((1,), (0,)), ((), ())),
                        preferred_element_type=_F32)
                    ps = jnp.sum(e, axis=1)
                    if l_run[b][h] is None:
                        l_run[b][h], acc[b][h] = ps, pv
                    else:
                        l_run[b][h] = l_run[b][h] + ps
                        acc[b][h] = acc[b][h] + pv

        cp.wait()
        process_slot(me)
        for o in (1, 3, 2):
            kv_rdmas[o - 1].wait()
            process_slot((me - o) % N_DEV)

        for b in range(B):
            for h in range(H_LOC):
                ctx_scr[b, :, h * DH:(h + 1) * DH] = (
                    acc[b][h] / l_run[b][h][:, None])

        for b in range(B):
            pb = lax.dot_general(ctx_scr[b], wo_ref[...],
                                 (((1,), (0,)), ((), ())),
                                 preferred_element_type=_F32)
            pb = pb.astype(_BF16)
            p_ref[2 * b] = pb[:CH]
            p_ref[2 * b + 1] = pb[CH:]

        if phase == "attn":
            for c in range(N_DEV):
                out_ref[c // 2, pl.ds((c % 2) * CH, CH), :] = p_ref[c].astype(_F32)
            return

        rs_rdmas = []
        for o in range(1, N_DEV):
            dst = (me + o) % N_DEV
            rdma = pltpu.make_async_remote_copy(
                src_ref=p_ref.at[dst], dst_ref=rs_buf.at[o - 1],
                send_sem=rs_send.at[o], recv_sem=rs_recv.at[o],
                device_id=(dst,), device_id_type=_MESH)
            rdma.start()
            rs_rdmas.append(rdma)
        for rdma in rs_rdmas:
            rdma.wait()
        o4[me] = (p_ref[me] + rs_buf[0]) + (rs_buf[1] + rs_buf[2])

        ag_rdmas = []
        for o in range(1, N_DEV):
            dst = (me + o) % N_DEV
            rdma = pltpu.make_async_remote_copy(
                src_ref=o4.at[me], dst_ref=o4.at[me],
                send_sem=ag_send.at[o], recv_sem=ag_rsem.at[o],
                device_id=(dst,), device_id_type=_MESH)
            rdma.start()
            ag_rdmas.append(rdma)
        for rdma in ag_rdmas:
            rdma.wait()

        for c in range(N_DEV):
            out_ref[c // 2, pl.ds((c % 2) * CH, CH), :] = o4[c].astype(_F32)

    vmem = pl.BlockSpec(memory_space=pltpu.MemorySpace.VMEM)
    hbm = pl.BlockSpec(memory_space=pltpu.MemorySpace.HBM)

    grid_kernel = pl.pallas_call(
        body,
        out_shape=jax.ShapeDtypeStruct((B, SQ, D_MODEL), _F32),
        in_specs=[vmem, vmem, vmem, vmem],
        out_specs=vmem,
        scratch_shapes=[
            pltpu.VMEM((N_DEV, H_LOC, B, SKV_LOC, 2 * DH), _BF16),
            pltpu.VMEM((B, SQ, HD_LOC), _F32),
            pltpu.VMEM((B, SQ, HD_LOC), _F32),
            pltpu.VMEM((N_DEV, CH, D_MODEL), _BF16),
            pltpu.VMEM((N_DEV - 1, CH, D_MODEL), _BF16),
            pltpu.VMEM((N_DEV, CH, D_MODEL), _BF16),
            pltpu.SemaphoreType.DMA,
            pltpu.SemaphoreType.DMA((N_DEV,)),
            pltpu.SemaphoreType.DMA((N_DEV,)),
            pltpu.SemaphoreType.DMA((N_DEV,)),
            pltpu.SemaphoreType.DMA((N_DEV,)),
            pltpu.SemaphoreType.DMA((N_DEV,)),
            pltpu.SemaphoreType.DMA((N_DEV,)),
        ],
        compiler_params=pltpu.CompilerParams(
            collective_id=collective_id,
            vmem_limit_bytes=60 * 1024 * 1024),
    )

    def kernel(x, Wq, K_ext, V_ext, Wo):
        k_t = jnp.transpose(K_ext, (2, 0, 1, 3))
        v_t = jnp.transpose(V_ext, (2, 0, 1, 3))
        kv = jnp.concatenate([k_t, v_t], axis=-1).astype(_BF16)
        return grid_kernel(x, Wq, kv, Wo)

    return kernel


kernel = _build("full")


# device time: 112169 ns/iter; 2.9262x vs baseline; 1.0172x over previous
import jax
import jax.numpy as jnp
from jax import lax
from jax.experimental import pallas as pl
from jax.experimental.pallas import tpu as pltpu

N_DEV = 4
B, SQ, D_MODEL = 2, 512, 768
HQ_TOTAL, DH = 32, 64
H_LOC = HQ_TOTAL // N_DEV
SKV_LOC = 512
SKV = N_DEV * SKV_LOC
HD_LOC = H_LOC * DH
ROWS = B * SQ
CH = ROWS // N_DEV
BLK = 64

_MESH = pl.DeviceIdType.MESH
_F32 = jnp.float32
_BF16 = jnp.bfloat16

PHASES = ("barrier", "a2a", "attn", "full")


def _build(phase="full", collective_id=0):
    assert phase in PHASES

    def body(x_ref, wq_ref, kv_ref, wo_ref, out_ref,
             kv_recv, q_scr, ctx_scr, p_ref, rs_buf, o4,
             cp_sem, kv_send, kv_rsem,
             rs_send, rs_recv, ag_send, ag_rsem):
        me = lax.axis_index("i")
        right = (me + 1) % N_DEV

        bsem = pltpu.get_barrier_semaphore()
        for o in range(1, N_DEV):
            pl.semaphore_signal(bsem, inc=1, device_id=((me + o) % N_DEV,),
                                device_id_type=_MESH)
        pl.semaphore_wait(bsem, N_DEV - 1)

        if phase == "barrier":
            out_ref[...] = jnp.zeros_like(out_ref)
            return

        cp = pltpu.make_async_copy(
            kv_ref.at[pl.ds(me * H_LOC, H_LOC)], kv_recv.at[me], cp_sem)
        cp.start()

        kv_rdmas = []
        for o in range(1, N_DEV):
            dst = (me + o) % N_DEV
            r = pltpu.make_async_remote_copy(
                src_ref=kv_ref.at[pl.ds(dst * H_LOC, H_LOC)],
                dst_ref=kv_recv.at[me],
                send_sem=kv_send.at[o], recv_sem=kv_rsem.at[o],
                device_id=(dst,), device_id_type=_MESH)
            r.start()
            kv_rdmas.append(r)

        for b in range(B):
            q_scr[b] = lax.dot_general(
                x_ref[b], wq_ref[...], (((1,), (0,)), ((), ())),
                preferred_element_type=_F32)

        if phase == "a2a":
            cp.wait()
            for r in kv_rdmas:
                r.wait()
            out_ref[...] = jnp.zeros_like(out_ref)
            out_ref[0, :, :DH] = kv_recv[0, 0, 0, :, :DH].astype(_F32)
            return

        qb0 = lax.broadcasted_iota(jnp.int32, (SQ, SKV_LOC), 0) // BLK
        kb0 = lax.broadcasted_iota(jnp.int32, (SQ, SKV_LOC), 1) // BLK
        mask_s = (qb0 % 4) == (kb0 % 4)

        l_run = [[None] * H_LOC for _ in range(B)]
        acc = [[None] * H_LOC for _ in range(B)]

        def process_slot(slot):
            for b in range(B):
                for h in range(H_LOC):
                    q = q_scr[b, :, h * DH:(h + 1) * DH].astype(_BF16)
                    k = kv_recv[slot, h, b, :, :DH]
                    v = kv_recv[slot, h, b, :, DH:]
                    sb = lax.dot_general(q, k, (((1,), (1,)), ((), ())),
                                         preferred_element_type=_F32) * 0.125
                    e = jnp.where(mask_s, jnp.exp(sb), 0.0)
                    pv = lax.dot_general(
                        e.astype(_BF16), v, (((1,), (0,)), ((), ())),
                        preferred_element_type=_F32)
                    ps = jnp.sum(e, axis=1)
                    if l_run[b][h] is None:
                        l_run[b][h], acc[b][h] = ps, pv
                    else:
                        l_run[b][h] = l_run[b][h] + ps
                        acc[b][h] = acc[b][h] + pv

        cp.wait()
        process_slot(me)
        for o in (1, 3, 2):
            kv_rdmas[o - 1].wait()
            process_slot((me - o) % N_DEV)

        for b in range(B):
            for h in range(H_LOC):
                ctx_scr[b, :, h * DH:(h + 1) * DH] = (
                    acc[b][h] / l_run[b][h][:, None])

        for b in range(B):
            pb = lax.dot_general(ctx_scr[b], wo_ref[...],
                                 (((1,), (0,)), ((), ())),
                                 preferred_element_type=_F32)
            pb = pb.astype(_BF16)
            p_ref[2 * b] = pb[:CH]
            p_ref[2 * b + 1] = pb[CH:]

        if phase == "attn":
            for c in range(N_DEV):
                out_ref[c // 2, pl.ds((c % 2) * CH, CH), :] = p_ref[c].astype(_F32)
            return

        rs_rdmas = []
        for o in range(1, N_DEV):
            dst = (me + o) % N_DEV
            rdma = pltpu.make_async_remote_copy(
                src_ref=p_ref.at[dst], dst_ref=rs_buf.at[o - 1],
                send_sem=rs_send.at[o], recv_sem=rs_recv.at[o],
                device_id=(dst,), device_id_type=_MESH)
            rdma.start()
            rs_rdmas.append(rdma)
        for rdma in rs_rdmas:
            rdma.wait()
        o4[me] = (p_ref[me] + rs_buf[0]) + (rs_buf[1] + rs_buf[2])

        ag_rdmas = []
        for o in range(1, N_DEV):
            dst = (me + o) % N_DEV
            rdma = pltpu.make_async_remote_copy(
                src_ref=o4.at[me], dst_ref=o4.at[me],
                send_sem=ag_send.at[o], recv_sem=ag_rsem.at[o],
                device_id=(dst,), device_id_type=_MESH)
            rdma.start()
            ag_rdmas.append(rdma)
        for rdma in ag_rdmas:
            rdma.wait()

        for c in range(N_DEV):
            out_ref[c // 2, pl.ds((c % 2) * CH, CH), :] = o4[c].astype(_F32)

    vmem = pl.BlockSpec(memory_space=pltpu.MemorySpace.VMEM)
    hbm = pl.BlockSpec(memory_space=pltpu.MemorySpace.HBM)

    grid_kernel = pl.pallas_call(
        body,
        out_shape=jax.ShapeDtypeStruct((B, SQ, D_MODEL), _F32),
        in_specs=[vmem, vmem, hbm, vmem],
        out_specs=vmem,
        scratch_shapes=[
            pltpu.VMEM((N_DEV, H_LOC, B, SKV_LOC, 2 * DH), _BF16),
            pltpu.VMEM((B, SQ, HD_LOC), _F32),
            pltpu.VMEM((B, SQ, HD_LOC), _F32),
            pltpu.VMEM((N_DEV, CH, D_MODEL), _BF16),
            pltpu.VMEM((N_DEV - 1, CH, D_MODEL), _BF16),
            pltpu.VMEM((N_DEV, CH, D_MODEL), _BF16),
            pltpu.SemaphoreType.DMA,
            pltpu.SemaphoreType.DMA((N_DEV,)),
            pltpu.SemaphoreType.DMA((N_DEV,)),
            pltpu.SemaphoreType.DMA((N_DEV,)),
            pltpu.SemaphoreType.DMA((N_DEV,)),
            pltpu.SemaphoreType.DMA((N_DEV,)),
            pltpu.SemaphoreType.DMA((N_DEV,)),
        ],
        compiler_params=pltpu.CompilerParams(
            collective_id=collective_id,
            vmem_limit_bytes=60 * 1024 * 1024),
    )

    def kernel(x, Wq, K_ext, V_ext, Wo):
        k_t = jnp.transpose(K_ext, (2, 0, 1, 3))
        v_t = jnp.transpose(V_ext, (2, 0, 1, 3))
        kv = jnp.concatenate([k_t, v_t], axis=-1).astype(_BF16)
        return grid_kernel(x, Wq, kv, Wo)

    return kernel


kernel = _build("full")
